# baseline (speedup 1.0000x reference)
import functools

import jax
import jax.numpy as jnp
from jax import lax
from jax.experimental import pallas as pl
from jax.experimental.pallas import tpu as pltpu
from jax.experimental.pallas import tpu_sc as plsc

_NC = 2
_NS = 16
_NW = _NC * _NS
_LANE = 128


def _flatten_copy(w):
    def body(in_ref, out_ref, sem):
        cp = pltpu.make_async_copy(in_ref.at[0, :], out_ref, sem)
        cp.start()
        cp.wait()

    wt = w.T
    return pl.pallas_call(
        body,
        in_specs=[pl.BlockSpec(memory_space=pl.ANY)],
        out_specs=pl.BlockSpec(memory_space=pl.ANY),
        out_shape=jax.ShapeDtypeStruct((w.shape[0],), jnp.float32),
        scratch_shapes=[pltpu.SemaphoreType.DMA],
    )(wt)


@functools.partial(jax.jit, static_argnums=(2,))
def _sc_gather(idx, table, chunks):
    mesh = plsc.VectorSubcoreMesh(core_axis_name="c", subcore_axis_name="s")

    @functools.partial(
        pl.kernel,
        out_type=jax.ShapeDtypeStruct((_NW, chunks, _LANE), jnp.float32),
        mesh=mesh,
        scratch_types=[
            pltpu.VMEM((chunks, _LANE), jnp.int32),
            pltpu.VMEM((chunks, _LANE), jnp.float32),
            pltpu.SemaphoreType.DMA((chunks,)),
            pltpu.SemaphoreType.DMA,
        ],
    )
    def run(idx_hbm, table_hbm, out_hbm, idx_v, rows_v, gsems, osem):
        wid = lax.axis_index("s") * _NC + lax.axis_index("c")
        pltpu.sync_copy(idx_hbm.at[wid], idx_v)
        gathers = [
            pltpu.async_copy(table_hbm.at[idx_v.at[j]], rows_v.at[j], gsems.at[j])
            for j in range(chunks)
        ]
        outs = []
        for j in range(chunks):
            gathers[j].wait()
            outs.append(pltpu.async_copy(rows_v.at[j], out_hbm.at[wid].at[j], osem))
        for cp in outs:
            cp.wait()

    return run(idx, table)


def kernel(inputs, weight):
    batch = inputs.shape[0]
    table = _flatten_copy(weight)
    per_w = -(-batch // _NW)
    chunks = -(-per_w // _LANE)
    batch_pad = _NW * chunks * _LANE
    idx = inputs.astype(jnp.int32)
    if batch_pad != batch:
        idx = jnp.pad(idx, (0, batch_pad - batch))
    idx = idx.reshape(_NW, chunks, _LANE)
    out = _sc_gather(idx, table, chunks)
    return out.reshape(batch_pad, 1)[:batch]
